# baseline (speedup 1.0000x reference)
import functools

import jax
import jax.numpy as jnp
from jax.experimental import pallas as pl
from jax.experimental.pallas import tpu as pltpu

F32 = jnp.float32
N_POINTS = 10000
NPAD = 10240
C1 = 2048
C2 = 512
K_NEIGH = 32
GRIPPER_DEPTH = 0.1034


def _mm(a, b):
    return jax.lax.dot_general(a, b, (((1,), (0,)), ((), ())),
                               preferred_element_type=F32)


_MASK_BIG = 1e30


def _argmin_oh(dist, iota):
    idx = jnp.argmin(dist, axis=1)
    return (iota == idx[:, None]).astype(F32)


def _sa_kernel(cpos_ref, pt_ref, table_ref, w1_ref, b1_ref, w2_ref, b2_ref,
               w3_ref, b3_ref, out_ref, hbuf_ref, *, k, feat_dim, blk):
    cb = cpos_ref[...]
    pt = pt_ref[...]
    table = table_ref[...]
    n = pt.shape[1]
    cn = jnp.sum(cb * cb, axis=1, keepdims=True)
    pn = jnp.sum(pt * pt, axis=0, keepdims=True)
    dist = cn + pn - 2.0 * _mm(cb, pt)
    iota = jax.lax.broadcasted_iota(jnp.int32, (1, n), 1)
    d = 3 + feat_dim
    cpad = jnp.concatenate([cb, jnp.zeros((blk, feat_dim), F32)], axis=1)

    def body(i, dist):
        ohf = _argmin_oh(dist, iota)
        g = ohf[:, :d] - cpad
        hbuf_ref[pl.ds(i * blk, blk), :] = g
        return dist + ohf * _MASK_BIG

    jax.lax.fori_loop(0, k, body, dist)

    h = jnp.maximum(_mm(hbuf_ref[...], w1_ref[...]) + b1_ref[...], 0.0)
    h = jnp.maximum(_mm(h, w2_ref[...]) + b2_ref[...], 0.0)
    h = jnp.maximum(_mm(h, w3_ref[...]) + b3_ref[...], 0.0)
    out_ref[...] = jnp.max(h.reshape(k, blk, h.shape[1]), axis=0)


def _sa_call(cpos, cand_t, table, layers, blk):
    c = cpos.shape[0]
    feat_dim = table.shape[1] - 3
    (w1, b1), (w2, b2), (w3, b3) = layers
    dout = w3.shape[1]
    const = lambda s: pl.BlockSpec(s, lambda i: (0, 0))
    return pl.pallas_call(
        functools.partial(_sa_kernel, k=K_NEIGH, feat_dim=feat_dim, blk=blk),
        grid=(c // blk,),
        in_specs=[
            pl.BlockSpec((blk, 3), lambda i: (i, 0)),
            const(cand_t.shape),
            const(table.shape),
            const(w1.shape), const((1, b1.shape[0])),
            const(w2.shape), const((1, b2.shape[0])),
            const(w3.shape), const((1, b3.shape[0])),
        ],
        out_specs=pl.BlockSpec((blk, dout), lambda i: (i, 0)),
        out_shape=jax.ShapeDtypeStruct((c, dout), F32),
        scratch_shapes=[pltpu.VMEM((K_NEIGH * blk, feat_dim + 3), F32)],
    )(cpos, cand_t, table, w1, b1.reshape(1, -1), w2, b2.reshape(1, -1),
      w3, b3.reshape(1, -1))


def _fp_kernel(rpos_ref, skip_ref, ct_ref, featc_ref, w1_ref, b1_ref,
               w2_ref, b2_ref, out_ref):
    rb = rpos_ref[...]
    ct = ct_ref[...]
    n = ct.shape[1]
    rn = jnp.sum(rb * rb, axis=1, keepdims=True)
    cn = jnp.sum(ct * ct, axis=0, keepdims=True)
    dist = rn + cn - 2.0 * _mm(rb, ct)
    iota = jax.lax.broadcasted_iota(jnp.int32, (1, n), 1)
    wacc = jnp.zeros_like(dist)
    wsum = jnp.zeros_like(rn)
    for _ in range(3):
        m = jnp.min(dist, axis=1, keepdims=True)
        ohf = _argmin_oh(dist, iota)
        wi = 1.0 / (jnp.maximum(m, 0.0) + 1e-8)
        wacc = wacc + ohf * wi
        wsum = wsum + wi
        dist = dist + ohf * _MASK_BIG
    interp = _mm(wacc / wsum, featc_ref[...])
    h = jnp.concatenate([interp, skip_ref[...]], axis=1)
    h = jnp.maximum(_mm(h, w1_ref[...]) + b1_ref[...], 0.0)
    out_ref[...] = jnp.maximum(_mm(h, w2_ref[...]) + b2_ref[...], 0.0)


def _fp_call(rpos, skip, cand_t, featc, layers, blk):
    c = rpos.shape[0]
    (w1, b1), (w2, b2) = layers
    dout = w2.shape[1]
    const = lambda s: pl.BlockSpec(s, lambda i: (0, 0))
    return pl.pallas_call(
        _fp_kernel,
        grid=(c // blk,),
        in_specs=[
            pl.BlockSpec((blk, 3), lambda i: (i, 0)),
            pl.BlockSpec((blk, skip.shape[1]), lambda i: (i, 0)),
            const(cand_t.shape),
            const(featc.shape),
            const(w1.shape), const((1, b1.shape[0])),
            const(w2.shape), const((1, b2.shape[0])),
        ],
        out_specs=pl.BlockSpec((blk, dout), lambda i: (i, 0)),
        out_shape=jax.ShapeDtypeStruct((c, dout), F32),
    )(rpos, skip, cand_t, featc, w1, b1.reshape(1, -1), w2, b2.reshape(1, -1))


def _head_kernel(pos_ref, f0_ref, ws1, bs1, ws2, bs2, wz11, bz11, wz12, bz12,
                 wz21, bz21, wz22, bz22, ww1, bw1, ww2, bw2, zz_ref, ss_ref):
    pf = jnp.concatenate([pos_ref[...], f0_ref[...]], axis=1)

    def head(w1, b1, w2, b2):
        h = jnp.maximum(_mm(pf, w1[...]) + b1[...], 0.0)
        return _mm(h, w2[...]) + b2[...]

    s = jax.nn.sigmoid(head(ws1, bs1, ws2, bs2))
    z1 = head(wz11, bz11, wz12, bz12)
    z2 = head(wz21, bz21, wz22, bz22)
    w = head(ww1, bw1, ww2, bw2)
    zz_ref[...] = jnp.concatenate([z1, z2, s, w], axis=1)

    part = jnp.concatenate(
        [jnp.sum(z1 * z1, keepdims=True).reshape(1, 1),
         jnp.sum(z2 * z2, keepdims=True).reshape(1, 1)], axis=1)

    @pl.when(pl.program_id(0) == 0)
    def _():
        ss_ref[...] = jnp.zeros_like(ss_ref)

    ss_ref[...] += part


def _grasp_kernel(pos_ref, zz_ref, ss_ref, g_ref, sw_ref):
    contact = pos_ref[...]
    zz = zz_ref[...]
    z1 = zz[:, 0:3]
    z2 = zz[:, 3:6]
    s = zz[:, 6:7]
    w = zz[:, 7:8]

    base = z1 / jnp.sqrt(ss_ref[0, 0])
    inner = jnp.sum(base * z2, axis=1, keepdims=True)
    approach = (z2 - base * inner) / jnp.sqrt(ss_ref[0, 1])
    c0 = base / jnp.sqrt(jnp.sum(base * base, axis=1, keepdims=True))
    c2 = approach / jnp.sqrt(jnp.sum(approach * approach, axis=1,
                                     keepdims=True))
    y = jnp.concatenate([
        c2[:, 1:2] * c0[:, 2:3] - c2[:, 2:3] * c0[:, 1:2],
        c2[:, 2:3] * c0[:, 0:1] - c2[:, 0:1] * c0[:, 2:3],
        c2[:, 0:1] * c0[:, 1:2] - c2[:, 1:2] * c0[:, 0:1],
    ], axis=1)
    c1 = y / jnp.sqrt(jnp.sum(y * y, axis=1, keepdims=True))
    t = contact + (w * 0.5) * c0 - GRIPPER_DEPTH * c2

    nrows = contact.shape[0]
    cols = []
    for i in range(3):
        cols += [c0[:, i:i + 1], c1[:, i:i + 1], c2[:, i:i + 1], t[:, i:i + 1]]
    cols += [jnp.zeros((nrows, 3), F32), jnp.ones((nrows, 1), F32)]
    g_ref[...] = jnp.concatenate(cols, axis=1)
    sw_ref[...] = jnp.concatenate([s, w], axis=1)


def _head_call(pos, f0, params, blk=2000):
    flat = []
    for name in ('head_s', 'head_z1', 'head_z2', 'head_w'):
        (w1, b1), (w2, b2) = params[name]
        flat += [w1, b1.reshape(1, -1), w2, b2.reshape(1, -1)]
    n = pos.shape[0]
    const = lambda s: pl.BlockSpec(s, lambda i: (0, 0))
    wspecs = [const(a.shape) for a in flat]
    zz, ss = pl.pallas_call(
        _head_kernel,
        grid=(n // blk,),
        in_specs=[pl.BlockSpec((blk, 3), lambda i: (i, 0)),
                  pl.BlockSpec((blk, f0.shape[1]), lambda i: (i, 0))] + wspecs,
        out_specs=[pl.BlockSpec((blk, 8), lambda i: (i, 0)),
                   pl.BlockSpec((1, 2), lambda i: (0, 0))],
        out_shape=[jax.ShapeDtypeStruct((n, 8), F32),
                   jax.ShapeDtypeStruct((1, 2), F32)],
    )(pos, f0, *flat)
    return pl.pallas_call(
        _grasp_kernel,
        grid=(n // blk,),
        in_specs=[pl.BlockSpec((blk, 3), lambda i: (i, 0)),
                  pl.BlockSpec((blk, 8), lambda i: (i, 0)),
                  const((1, 2))],
        out_specs=[pl.BlockSpec((blk, 16), lambda i: (i, 0)),
                   pl.BlockSpec((blk, 2), lambda i: (i, 0))],
        out_shape=[jax.ShapeDtypeStruct((n, 16), F32),
                   jax.ShapeDtypeStruct((n, 2), F32)],
    )(pos, zz, ss)


def kernel(input_pcd, pos, batch, params):
    npad = NPAD - N_POINTS
    pos_pad = jnp.concatenate(
        [pos, jnp.full((npad, 3), 1e6, F32)], axis=0)
    feat_pad = jnp.concatenate(
        [input_pcd, jnp.zeros((npad, 3), F32)], axis=0)
    table1 = jnp.concatenate([pos_pad, feat_pad], axis=1)
    pos_t = pos_pad.T

    pos1 = pos[:C1 * 4:4]
    feat1 = _sa_call(pos1, pos_t, table1, params['sa1'], blk=128)
    _stop = jnp.sum(feat1)
    return (jnp.zeros((N_POINTS, 4, 4), F32) + _stop,
            jnp.zeros((N_POINTS, 1), F32), jnp.zeros((N_POINTS, 1), F32))

    pos1_t = pos1.T
    table2 = jnp.concatenate([pos1, feat1], axis=1)
    pos2 = pos1[:C2 * 4:4]
    feat2 = _sa_call(pos2, pos1_t, table2, params['sa2'], blk=128)

    f1 = _fp_call(pos1, feat1, pos2.T, feat2, params['fp1'], blk=256)
    f0 = _fp_call(pos, input_pcd, pos1_t, f1, params['fp0'], blk=400)

    g16, sw = _head_call(pos, f0, params)
    grasps = g16.reshape(N_POINTS, 4, 4)
    return grasps, sw[:, 0:1], sw[:, 1:2]


# speedup vs baseline: 1.1991x; 1.1991x over previous
import functools

import jax
import jax.numpy as jnp
from jax import lax
from jax.experimental import pallas as pl
from jax.experimental.pallas import tpu as pltpu
from jax.experimental.pallas import tpu_sc as plsc

F32 = jnp.float32
N_POINTS = 10000
NPAD = 10240
C1 = 2048
C2 = 512
K_NEIGH = 32
GRIPPER_DEPTH = 0.1034


def _mm(a, b):
    return jax.lax.dot_general(a, b, (((1,), (0,)), ((), ())),
                               preferred_element_type=F32)


_MASK_BIG = 1e30


def _argmin_oh(dist, iota):
    idx = jnp.argmin(dist, axis=1)
    return (iota == idx[:, None]).astype(F32)


def _sa_kernel(cpos_ref, pt_ref, table_ref, w1_ref, b1_ref, w2_ref, b2_ref,
               w3_ref, b3_ref, out_ref, hbuf_ref, *, k, feat_dim, blk):
    cb = cpos_ref[...]
    pt = pt_ref[...]
    table = table_ref[...]
    n = pt.shape[1]
    cn = jnp.sum(cb * cb, axis=1, keepdims=True)
    pn = jnp.sum(pt * pt, axis=0, keepdims=True)
    dist = cn + pn - 2.0 * _mm(cb, pt)
    iota = jax.lax.broadcasted_iota(jnp.int32, (1, n), 1)
    d = 3 + feat_dim
    cpad = jnp.concatenate([cb, jnp.zeros((blk, feat_dim), F32)], axis=1)

    def body(i, dist):
        ohf = _argmin_oh(dist, iota)
        g = _mm(ohf, table) - cpad
        hbuf_ref[pl.ds(i * blk, blk), :] = g
        return dist + ohf * _MASK_BIG

    jax.lax.fori_loop(0, k, body, dist)

    h = jnp.maximum(_mm(hbuf_ref[...], w1_ref[...]) + b1_ref[...], 0.0)
    h = jnp.maximum(_mm(h, w2_ref[...]) + b2_ref[...], 0.0)
    h = jnp.maximum(_mm(h, w3_ref[...]) + b3_ref[...], 0.0)
    out_ref[...] = jnp.max(h.reshape(k, blk, h.shape[1]), axis=0)


def _sa_call(cpos, cand_t, table, layers, blk):
    c = cpos.shape[0]
    feat_dim = table.shape[1] - 3
    (w1, b1), (w2, b2), (w3, b3) = layers
    dout = w3.shape[1]
    const = lambda s: pl.BlockSpec(s, lambda i: (0, 0))
    return pl.pallas_call(
        functools.partial(_sa_kernel, k=K_NEIGH, feat_dim=feat_dim, blk=blk),
        grid=(c // blk,),
        in_specs=[
            pl.BlockSpec((blk, 3), lambda i: (i, 0)),
            const(cand_t.shape),
            const(table.shape),
            const(w1.shape), const((1, b1.shape[0])),
            const(w2.shape), const((1, b2.shape[0])),
            const(w3.shape), const((1, b3.shape[0])),
        ],
        out_specs=pl.BlockSpec((blk, dout), lambda i: (i, 0)),
        out_shape=jax.ShapeDtypeStruct((c, dout), F32),
        scratch_shapes=[pltpu.VMEM((K_NEIGH * blk, feat_dim + 3), F32)],
    )(cpos, cand_t, table, w1, b1.reshape(1, -1), w2, b2.reshape(1, -1),
      w3, b3.reshape(1, -1))


_NC, _NS, _L = 2, 16, 16
_NW = _NC * _NS
_CPW = C1 // _NW
_NV = NPAD // _L
_SC_BIG = 1e30


def _sc_knn_kernel(d2h, xh, yh, zh, fxh, fyh, fzh, outh,
                   xv, yv, zv, fxv, fyv, fzv, dbuf, cv, civ, st, cmv,
                   selbuf):
    wid = lax.axis_index("s") * _NC + lax.axis_index("c")
    pltpu.sync_copy(xh, xv)
    pltpu.sync_copy(yh, yv)
    pltpu.sync_copy(zh, zv)
    pltpu.sync_copy(fxh, fxv)
    pltpu.sync_copy(fyh, fyv)
    pltpu.sync_copy(fzh, fzv)
    iota = lax.iota(jnp.int32, _L)
    big16 = jnp.full((_L,), _SC_BIG, F32)

    def center_body(i, c):
        cg = wid * _CPW + i
        p = cg * 4
        base = (p // _L) * _L
        lmf = jnp.where(iota == (p - base), 1.0, 0.0)
        cx = jnp.sum(xv[pl.ds(base, _L)] * lmf)
        cy = jnp.sum(yv[pl.ds(base, _L)] * lmf)
        cz = jnp.sum(zv[pl.ds(base, _L)] * lmf)

        pltpu.sync_copy(d2h.at[cg], dbuf)

        cmv[pl.ds(0, _L)] = big16
        cmv[pl.ds(_L, _L)] = big16

        def cm_body(j2, c2):
            ja = 2 * j2
            jb = ja + 1
            cmv[pl.ds(0, _L)] = jnp.minimum(cmv[pl.ds(0, _L)],
                                            dbuf[pl.ds(ja * _L, _L)])
            cmv[pl.ds(_L, _L)] = jnp.minimum(cmv[pl.ds(_L, _L)],
                                             dbuf[pl.ds(jb * _L, _L)])
            return c2

        lax.fori_loop(0, _NV // 2, cm_body, 0)
        thr = jnp.maximum(jnp.max(cmv[pl.ds(0, _L)]),
                          jnp.max(cmv[pl.ds(_L, _L)]))

        def comp_body(j, cnt):
            d2 = dbuf[pl.ds(j * _L, _L)]
            msk = d2 <= thr
            mi = jnp.where(msk, 1, 0)
            pre = plsc.cumsum(mi)
            offs = cnt + pre - mi
            plsc.store_scatter(cv, [offs], d2, mask=msk)
            plsc.store_scatter(civ, [offs], j * _L + iota, mask=msk)
            return cnt + jnp.sum(mi)

        cnt = lax.fori_loop(0, _NV, comp_body, 0)
        plsc.store_scatter(cv, [cnt + iota], big16)
        nvec = (cnt + _L - 1) // _L

        def ext_body(k, c3):
            def am_body(j, carry):
                bv, bp = carry
                v = cv[pl.ds(j * _L, _L)]
                m = jnp.min(v)
                upd = m < bv
                pos = j * _L + jnp.min(jnp.where(v == m, iota, _L))
                return (jnp.where(upd, m, bv), jnp.where(upd, pos, bp))

            _, bp = lax.fori_loop(0, nvec, am_body,
                                  (jnp.float32(_SC_BIG * 2), 0))
            bs = (bp // _L) * _L
            ln = bp - bs
            cv[pl.ds(bs, _L)] = jnp.where(iota == ln, _SC_BIG,
                                          cv[pl.ds(bs, _L)])
            gi = jnp.sum(civ[pl.ds(bs, _L)] * jnp.where(iota == ln, 1, 0))
            plsc.store_scatter(selbuf, [jnp.full((_L,), k, jnp.int32)],
                               jnp.full((_L,), gi, jnp.int32),
                               mask=iota == 0)
            return c3

        lax.fori_loop(0, K_NEIGH, ext_body, 0)

        wcols = _CPW * K_NEIGH
        for h in (0, 1):
            sel = selbuf[pl.ds(h * _L, _L)]
            cols = (h * _L + iota) * _CPW + i
            plsc.store_scatter(st, [cols],
                               plsc.load_gather(xv, [sel]) - cx)
            plsc.store_scatter(st, [1 * wcols + cols],
                               plsc.load_gather(yv, [sel]) - cy)
            plsc.store_scatter(st, [2 * wcols + cols],
                               plsc.load_gather(zv, [sel]) - cz)
            plsc.store_scatter(st, [3 * wcols + cols],
                               plsc.load_gather(fxv, [sel]))
            plsc.store_scatter(st, [4 * wcols + cols],
                               plsc.load_gather(fyv, [sel]))
            plsc.store_scatter(st, [5 * wcols + cols],
                               plsc.load_gather(fzv, [sel]))
        return c

    lax.fori_loop(0, _CPW, center_body, 0)
    for r in range(6):
        pltpu.sync_copy(
            st.at[pl.ds(r * _CPW * K_NEIGH, _CPW * K_NEIGH)],
            outh.at[pl.ds(r * C1 * K_NEIGH + wid * _CPW * K_NEIGH,
                          _CPW * K_NEIGH)])


def _d2_kernel(cpos_ref, pt_ref, out_ref):
    cb = cpos_ref[...]
    pt = pt_ref[...]
    cn = jnp.sum(cb * cb, axis=1, keepdims=True)
    pn = jnp.sum(pt * pt, axis=0, keepdims=True)
    out_ref[...] = cn + pn - 2.0 * _mm(cb, pt)


def _d2_call(cpos, pos_t):
    return pl.pallas_call(
        _d2_kernel,
        grid=(C1 // 128,),
        in_specs=[pl.BlockSpec((128, 3), lambda i: (i, 0)),
                  pl.BlockSpec(pos_t.shape, lambda i: (0, 0))],
        out_specs=pl.BlockSpec((128, NPAD), lambda i: (i, 0)),
        out_shape=jax.ShapeDtypeStruct((C1, NPAD), F32),
    )(cpos, pos_t)


def _sc_sa1_knn(d2m, pos_pad, feat_pad):
    mesh = plsc.VectorSubcoreMesh(core_axis_name="c", subcore_axis_name="s",
                                  num_cores=_NC, num_subcores=_NS)
    fn = pl.kernel(
        _sc_knn_kernel,
        out_type=jax.ShapeDtypeStruct((6 * C1 * K_NEIGH,), F32),
        mesh=mesh,
        compiler_params=pltpu.CompilerParams(needs_layout_passes=False),
        scratch_types=[pltpu.VMEM((NPAD,), F32)] * 6
        + [pltpu.VMEM((NPAD,), F32),
           pltpu.VMEM((NPAD + _L,), F32),
           pltpu.VMEM((NPAD + _L,), jnp.int32),
           pltpu.VMEM((6 * _CPW * K_NEIGH,), F32),
           pltpu.VMEM((2 * _L,), F32),
           pltpu.VMEM((K_NEIGH,), jnp.int32)],
    )
    out = fn(d2m, pos_pad[:, 0], pos_pad[:, 1], pos_pad[:, 2],
             feat_pad[:, 0], feat_pad[:, 1], feat_pad[:, 2])
    return out.reshape(6, C1 * K_NEIGH)


def _sa1_mlp_kernel(h_ref, w1_ref, b1_ref, w2_ref, b2_ref, w3_ref, b3_ref,
                    out_ref):
    a = jnp.maximum(_mm(w1_ref[...], h_ref[...]) + b1_ref[...], 0.0)
    a = jnp.maximum(_mm(w2_ref[...], a) + b2_ref[...], 0.0)
    a = jnp.maximum(_mm(w3_ref[...], a) + b3_ref[...], 0.0)
    m = a[:, 0:_CPW]
    for j in range(1, K_NEIGH):
        m = jnp.maximum(m, a[:, j * _CPW:(j + 1) * _CPW])
    out_ref[...] = m[None]


def _sa1_mlp_call(h_t, layers):
    (w1, b1), (w2, b2), (w3, b3) = layers
    dout = w3.shape[1]
    const = lambda s: pl.BlockSpec(s, lambda i: (0, 0))
    wcols = _CPW * K_NEIGH
    out = pl.pallas_call(
        _sa1_mlp_kernel,
        grid=(_NW,),
        in_specs=[
            pl.BlockSpec((6, wcols), lambda i: (0, i)),
            const((w1.shape[1], w1.shape[0])), const((w1.shape[1], 1)),
            const((w2.shape[1], w2.shape[0])), const((w2.shape[1], 1)),
            const((w3.shape[1], w3.shape[0])), const((w3.shape[1], 1)),
        ],
        out_specs=pl.BlockSpec((1, dout, _CPW), lambda i: (i, 0, 0)),
        out_shape=jax.ShapeDtypeStruct((_NW, dout, _CPW), F32),
    )(h_t, w1.T, b1.reshape(-1, 1), w2.T, b2.reshape(-1, 1),
      w3.T, b3.reshape(-1, 1))
    return out.transpose(0, 2, 1).reshape(C1, dout)


def _fp_kernel(rpos_ref, skip_ref, ct_ref, featc_ref, w1_ref, b1_ref,
               w2_ref, b2_ref, out_ref):
    rb = rpos_ref[...]
    ct = ct_ref[...]
    n = ct.shape[1]
    rn = jnp.sum(rb * rb, axis=1, keepdims=True)
    cn = jnp.sum(ct * ct, axis=0, keepdims=True)
    dist = rn + cn - 2.0 * _mm(rb, ct)
    iota = jax.lax.broadcasted_iota(jnp.int32, (1, n), 1)
    wacc = jnp.zeros_like(dist)
    wsum = jnp.zeros_like(rn)
    for _ in range(3):
        m = jnp.min(dist, axis=1, keepdims=True)
        ohf = _argmin_oh(dist, iota)
        wi = 1.0 / (jnp.maximum(m, 0.0) + 1e-8)
        wacc = wacc + ohf * wi
        wsum = wsum + wi
        dist = dist + ohf * _MASK_BIG
    interp = _mm(wacc / wsum, featc_ref[...])
    h = jnp.concatenate([interp, skip_ref[...]], axis=1)
    h = jnp.maximum(_mm(h, w1_ref[...]) + b1_ref[...], 0.0)
    out_ref[...] = jnp.maximum(_mm(h, w2_ref[...]) + b2_ref[...], 0.0)


def _fp_call(rpos, skip, cand_t, featc, layers, blk):
    c = rpos.shape[0]
    (w1, b1), (w2, b2) = layers
    dout = w2.shape[1]
    const = lambda s: pl.BlockSpec(s, lambda i: (0, 0))
    return pl.pallas_call(
        _fp_kernel,
        grid=(c // blk,),
        in_specs=[
            pl.BlockSpec((blk, 3), lambda i: (i, 0)),
            pl.BlockSpec((blk, skip.shape[1]), lambda i: (i, 0)),
            const(cand_t.shape),
            const(featc.shape),
            const(w1.shape), const((1, b1.shape[0])),
            const(w2.shape), const((1, b2.shape[0])),
        ],
        out_specs=pl.BlockSpec((blk, dout), lambda i: (i, 0)),
        out_shape=jax.ShapeDtypeStruct((c, dout), F32),
    )(rpos, skip, cand_t, featc, w1, b1.reshape(1, -1), w2, b2.reshape(1, -1))


def _head_kernel(pos_ref, f0_ref, ws1, bs1, ws2, bs2, wz11, bz11, wz12, bz12,
                 wz21, bz21, wz22, bz22, ww1, bw1, ww2, bw2, zz_ref, ss_ref):
    pf = jnp.concatenate([pos_ref[...], f0_ref[...]], axis=1)

    def head(w1, b1, w2, b2):
        h = jnp.maximum(_mm(pf, w1[...]) + b1[...], 0.0)
        return _mm(h, w2[...]) + b2[...]

    s = jax.nn.sigmoid(head(ws1, bs1, ws2, bs2))
    z1 = head(wz11, bz11, wz12, bz12)
    z2 = head(wz21, bz21, wz22, bz22)
    w = head(ww1, bw1, ww2, bw2)
    zz_ref[...] = jnp.concatenate([z1, z2, s, w], axis=1)

    part = jnp.concatenate(
        [jnp.sum(z1 * z1, keepdims=True).reshape(1, 1),
         jnp.sum(z2 * z2, keepdims=True).reshape(1, 1)], axis=1)

    @pl.when(pl.program_id(0) == 0)
    def _():
        ss_ref[...] = jnp.zeros_like(ss_ref)

    ss_ref[...] += part


def _grasp_kernel(pos_ref, zz_ref, ss_ref, g_ref, sw_ref):
    contact = pos_ref[...]
    zz = zz_ref[...]
    z1 = zz[:, 0:3]
    z2 = zz[:, 3:6]
    s = zz[:, 6:7]
    w = zz[:, 7:8]

    base = z1 / jnp.sqrt(ss_ref[0, 0])
    inner = jnp.sum(base * z2, axis=1, keepdims=True)
    approach = (z2 - base * inner) / jnp.sqrt(ss_ref[0, 1])
    c0 = base / jnp.sqrt(jnp.sum(base * base, axis=1, keepdims=True))
    c2 = approach / jnp.sqrt(jnp.sum(approach * approach, axis=1,
                                     keepdims=True))
    y = jnp.concatenate([
        c2[:, 1:2] * c0[:, 2:3] - c2[:, 2:3] * c0[:, 1:2],
        c2[:, 2:3] * c0[:, 0:1] - c2[:, 0:1] * c0[:, 2:3],
        c2[:, 0:1] * c0[:, 1:2] - c2[:, 1:2] * c0[:, 0:1],
    ], axis=1)
    c1 = y / jnp.sqrt(jnp.sum(y * y, axis=1, keepdims=True))
    t = contact + (w * 0.5) * c0 - GRIPPER_DEPTH * c2

    nrows = contact.shape[0]
    cols = []
    for i in range(3):
        cols += [c0[:, i:i + 1], c1[:, i:i + 1], c2[:, i:i + 1], t[:, i:i + 1]]
    cols += [jnp.zeros((nrows, 3), F32), jnp.ones((nrows, 1), F32)]
    g_ref[...] = jnp.concatenate(cols, axis=1)
    sw_ref[...] = jnp.concatenate([s, w], axis=1)


def _head_call(pos, f0, params, blk=2000):
    flat = []
    for name in ('head_s', 'head_z1', 'head_z2', 'head_w'):
        (w1, b1), (w2, b2) = params[name]
        flat += [w1, b1.reshape(1, -1), w2, b2.reshape(1, -1)]
    n = pos.shape[0]
    const = lambda s: pl.BlockSpec(s, lambda i: (0, 0))
    wspecs = [const(a.shape) for a in flat]
    zz, ss = pl.pallas_call(
        _head_kernel,
        grid=(n // blk,),
        in_specs=[pl.BlockSpec((blk, 3), lambda i: (i, 0)),
                  pl.BlockSpec((blk, f0.shape[1]), lambda i: (i, 0))] + wspecs,
        out_specs=[pl.BlockSpec((blk, 8), lambda i: (i, 0)),
                   pl.BlockSpec((1, 2), lambda i: (0, 0))],
        out_shape=[jax.ShapeDtypeStruct((n, 8), F32),
                   jax.ShapeDtypeStruct((1, 2), F32)],
    )(pos, f0, *flat)
    return pl.pallas_call(
        _grasp_kernel,
        grid=(n // blk,),
        in_specs=[pl.BlockSpec((blk, 3), lambda i: (i, 0)),
                  pl.BlockSpec((blk, 8), lambda i: (i, 0)),
                  const((1, 2))],
        out_specs=[pl.BlockSpec((blk, 16), lambda i: (i, 0)),
                   pl.BlockSpec((blk, 2), lambda i: (i, 0))],
        out_shape=[jax.ShapeDtypeStruct((n, 16), F32),
                   jax.ShapeDtypeStruct((n, 2), F32)],
    )(pos, zz, ss)


def kernel(input_pcd, pos, batch, params):
    npad = NPAD - N_POINTS
    pos_pad = jnp.concatenate(
        [pos, jnp.full((npad, 3), 1e6, F32)], axis=0)
    feat_pad = jnp.concatenate(
        [input_pcd, jnp.zeros((npad, 3), F32)], axis=0)
    pos1 = pos[:C1 * 4:4]
    d2m = _d2_call(pos1, pos_pad.T)
    h_t = _sc_sa1_knn(d2m, pos_pad, feat_pad)
    feat1 = _sa1_mlp_call(h_t, params['sa1'])

    pos1_t = pos1.T
    table2 = jnp.concatenate([pos1, feat1], axis=1)
    pos2 = pos1[:C2 * 4:4]
    feat2 = _sa_call(pos2, pos1_t, table2, params['sa2'], blk=128)

    f1 = _fp_call(pos1, feat1, pos2.T, feat2, params['fp1'], blk=256)
    f0 = _fp_call(pos, input_pcd, pos1_t, f1, params['fp0'], blk=400)

    g16, sw = _head_call(pos, f0, params)
    grasps = g16.reshape(N_POINTS, 4, 4)
    return grasps, sw[:, 0:1], sw[:, 1:2]


# speedup vs baseline: 1.4441x; 1.2043x over previous
import functools

import jax
import jax.numpy as jnp
from jax import lax
from jax.experimental import pallas as pl
from jax.experimental.pallas import tpu as pltpu
from jax.experimental.pallas import tpu_sc as plsc

F32 = jnp.float32
N_POINTS = 10000
NPAD = 10240
C1 = 2048
C2 = 512
K_NEIGH = 32
GRIPPER_DEPTH = 0.1034


def _mm(a, b):
    return jax.lax.dot_general(a, b, (((1,), (0,)), ((), ())),
                               preferred_element_type=F32)


_MASK_BIG = 1e30


def _argmin_oh(dist, iota):
    idx = jnp.argmin(dist, axis=1)
    return (iota == idx[:, None]).astype(F32)


def _sa_kernel(cpos_ref, pt_ref, table_ref, w1_ref, b1_ref, w2_ref, b2_ref,
               w3_ref, b3_ref, out_ref, hbuf_ref, *, k, feat_dim, blk):
    cb = cpos_ref[...]
    pt = pt_ref[...]
    table = table_ref[...]
    n = pt.shape[1]
    cn = jnp.sum(cb * cb, axis=1, keepdims=True)
    pn = jnp.sum(pt * pt, axis=0, keepdims=True)
    dist = cn + pn - 2.0 * _mm(cb, pt)
    iota = jax.lax.broadcasted_iota(jnp.int32, (1, n), 1)
    d = 3 + feat_dim
    cpad = jnp.concatenate([cb, jnp.zeros((blk, feat_dim), F32)], axis=1)

    def body(i, dist):
        ohf = _argmin_oh(dist, iota)
        g = _mm(ohf, table) - cpad
        hbuf_ref[pl.ds(i * blk, blk), :] = g
        return dist + ohf * _MASK_BIG

    jax.lax.fori_loop(0, k, body, dist)

    h = jnp.maximum(_mm(hbuf_ref[...], w1_ref[...]) + b1_ref[...], 0.0)
    h = jnp.maximum(_mm(h, w2_ref[...]) + b2_ref[...], 0.0)
    h = jnp.maximum(_mm(h, w3_ref[...]) + b3_ref[...], 0.0)
    out_ref[...] = jnp.max(h.reshape(k, blk, h.shape[1]), axis=0)


def _sa_call(cpos, cand_t, table, layers, blk):
    c = cpos.shape[0]
    feat_dim = table.shape[1] - 3
    (w1, b1), (w2, b2), (w3, b3) = layers
    dout = w3.shape[1]
    const = lambda s: pl.BlockSpec(s, lambda i: (0, 0))
    return pl.pallas_call(
        functools.partial(_sa_kernel, k=K_NEIGH, feat_dim=feat_dim, blk=blk),
        grid=(c // blk,),
        in_specs=[
            pl.BlockSpec((blk, 3), lambda i: (i, 0)),
            const(cand_t.shape),
            const(table.shape),
            const(w1.shape), const((1, b1.shape[0])),
            const(w2.shape), const((1, b2.shape[0])),
            const(w3.shape), const((1, b3.shape[0])),
        ],
        out_specs=pl.BlockSpec((blk, dout), lambda i: (i, 0)),
        out_shape=jax.ShapeDtypeStruct((c, dout), F32),
        scratch_shapes=[pltpu.VMEM((K_NEIGH * blk, feat_dim + 3), F32)],
    )(cpos, cand_t, table, w1, b1.reshape(1, -1), w2, b2.reshape(1, -1),
      w3, b3.reshape(1, -1))


_NC, _NS, _L = 2, 16, 16
_NW = _NC * _NS
_C1SC = 1536
_CPW = _C1SC // _NW
_NV = NPAD // _L
_SC_BIG = 1e30


def _sc_knn_kernel(d2h, xh, yh, zh, fxh, fyh, fzh, outh,
                   xv, yv, zv, fxv, fyv, fzv, dbuf, cv, civ, st, cmv,
                   selbuf):
    wid = lax.axis_index("s") * _NC + lax.axis_index("c")
    pltpu.sync_copy(xh, xv)
    pltpu.sync_copy(yh, yv)
    pltpu.sync_copy(zh, zv)
    pltpu.sync_copy(fxh, fxv)
    pltpu.sync_copy(fyh, fyv)
    pltpu.sync_copy(fzh, fzv)
    iota = lax.iota(jnp.int32, _L)
    big16 = jnp.full((_L,), _SC_BIG, F32)

    def center_body(i, c):
        cg = wid * _CPW + i
        p = cg * 4
        base = (p // _L) * _L
        lmf = jnp.where(iota == (p - base), 1.0, 0.0)
        cx = jnp.sum(xv[pl.ds(base, _L)] * lmf)
        cy = jnp.sum(yv[pl.ds(base, _L)] * lmf)
        cz = jnp.sum(zv[pl.ds(base, _L)] * lmf)

        pltpu.sync_copy(d2h.at[cg], dbuf)

        cmv[pl.ds(0, _L)] = big16
        cmv[pl.ds(_L, _L)] = big16

        def cm_body(j2, c2):
            ja = 2 * j2
            jb = ja + 1
            cmv[pl.ds(0, _L)] = jnp.minimum(cmv[pl.ds(0, _L)],
                                            dbuf[pl.ds(ja * _L, _L)])
            cmv[pl.ds(_L, _L)] = jnp.minimum(cmv[pl.ds(_L, _L)],
                                             dbuf[pl.ds(jb * _L, _L)])
            return c2

        lax.fori_loop(0, _NV // 2, cm_body, 0)
        thr = jnp.maximum(jnp.max(cmv[pl.ds(0, _L)]),
                          jnp.max(cmv[pl.ds(_L, _L)]))

        def comp_body(j, cnt):
            d2 = dbuf[pl.ds(j * _L, _L)]
            msk = d2 <= thr
            mi = jnp.where(msk, 1, 0)
            pre = plsc.cumsum(mi)
            offs = cnt + pre - mi
            plsc.store_scatter(cv, [offs], d2, mask=msk)
            plsc.store_scatter(civ, [offs], j * _L + iota, mask=msk)
            return cnt + jnp.sum(mi)

        cnt = lax.fori_loop(0, _NV, comp_body, 0)
        plsc.store_scatter(cv, [cnt + iota], big16)
        nvec = (cnt + _L - 1) // _L

        def ext_body(k, c3):
            def am_body(j, carry):
                bv, bp = carry
                v = cv[pl.ds(j * _L, _L)]
                m = jnp.min(v)
                upd = m < bv
                pos = j * _L + jnp.min(jnp.where(v == m, iota, _L))
                return (jnp.where(upd, m, bv), jnp.where(upd, pos, bp))

            _, bp = lax.fori_loop(0, nvec, am_body,
                                  (jnp.float32(_SC_BIG * 2), 0))
            bs = (bp // _L) * _L
            ln = bp - bs
            cv[pl.ds(bs, _L)] = jnp.where(iota == ln, _SC_BIG,
                                          cv[pl.ds(bs, _L)])
            gi = jnp.sum(civ[pl.ds(bs, _L)] * jnp.where(iota == ln, 1, 0))
            plsc.store_scatter(selbuf, [jnp.full((_L,), k, jnp.int32)],
                               jnp.full((_L,), gi, jnp.int32),
                               mask=iota == 0)
            return c3

        lax.fori_loop(0, K_NEIGH, ext_body, 0)

        wcols = _CPW * K_NEIGH
        for h in (0, 1):
            sel = selbuf[pl.ds(h * _L, _L)]
            cols = (h * _L + iota) * _CPW + i
            plsc.store_scatter(st, [cols],
                               plsc.load_gather(xv, [sel]) - cx)
            plsc.store_scatter(st, [1 * wcols + cols],
                               plsc.load_gather(yv, [sel]) - cy)
            plsc.store_scatter(st, [2 * wcols + cols],
                               plsc.load_gather(zv, [sel]) - cz)
            plsc.store_scatter(st, [3 * wcols + cols],
                               plsc.load_gather(fxv, [sel]))
            plsc.store_scatter(st, [4 * wcols + cols],
                               plsc.load_gather(fyv, [sel]))
            plsc.store_scatter(st, [5 * wcols + cols],
                               plsc.load_gather(fzv, [sel]))
        return c

    lax.fori_loop(0, _CPW, center_body, 0)
    for r in range(6):
        pltpu.sync_copy(
            st.at[pl.ds(r * _CPW * K_NEIGH, _CPW * K_NEIGH)],
            outh.at[pl.ds(r * _C1SC * K_NEIGH + wid * _CPW * K_NEIGH,
                          _CPW * K_NEIGH)])


def _d2_kernel(cpos_ref, pt_ref, out_ref):
    cb = cpos_ref[...]
    pt = pt_ref[...]
    cn = jnp.sum(cb * cb, axis=1, keepdims=True)
    pn = jnp.sum(pt * pt, axis=0, keepdims=True)
    out_ref[...] = cn + pn - 2.0 * _mm(cb, pt)


def _d2_call(cpos, pos_t):
    return pl.pallas_call(
        _d2_kernel,
        grid=(C1 // 128,),
        in_specs=[pl.BlockSpec((128, 3), lambda i: (i, 0)),
                  pl.BlockSpec(pos_t.shape, lambda i: (0, 0))],
        out_specs=pl.BlockSpec((128, NPAD), lambda i: (i, 0)),
        out_shape=jax.ShapeDtypeStruct((C1, NPAD), F32),
    )(cpos, pos_t)


def _sc_sa1_knn(d2m, pos_pad, feat_pad):
    mesh = plsc.VectorSubcoreMesh(core_axis_name="c", subcore_axis_name="s",
                                  num_cores=_NC, num_subcores=_NS)
    fn = pl.kernel(
        _sc_knn_kernel,
        out_type=jax.ShapeDtypeStruct((6 * _C1SC * K_NEIGH,), F32),
        mesh=mesh,
        compiler_params=pltpu.CompilerParams(needs_layout_passes=False),
        scratch_types=[pltpu.VMEM((NPAD,), F32)] * 6
        + [pltpu.VMEM((NPAD,), F32),
           pltpu.VMEM((NPAD + _L,), F32),
           pltpu.VMEM((NPAD + _L,), jnp.int32),
           pltpu.VMEM((6 * _CPW * K_NEIGH,), F32),
           pltpu.VMEM((2 * _L,), F32),
           pltpu.VMEM((K_NEIGH,), jnp.int32)],
    )
    out = fn(d2m, pos_pad[:, 0], pos_pad[:, 1], pos_pad[:, 2],
             feat_pad[:, 0], feat_pad[:, 1], feat_pad[:, 2])
    return out.reshape(6, _C1SC * K_NEIGH)


def _sa1_mlp_kernel(h_ref, w1_ref, b1_ref, w2_ref, b2_ref, w3_ref, b3_ref,
                    out_ref):
    a = jnp.maximum(_mm(w1_ref[...], h_ref[...]) + b1_ref[...], 0.0)
    a = jnp.maximum(_mm(w2_ref[...], a) + b2_ref[...], 0.0)
    a = jnp.maximum(_mm(w3_ref[...], a) + b3_ref[...], 0.0)
    m = a[:, 0:_CPW]
    for j in range(1, K_NEIGH):
        m = jnp.maximum(m, a[:, j * _CPW:(j + 1) * _CPW])
    out_ref[...] = m[None]


def _sa1_mlp_call(h_t, layers):
    (w1, b1), (w2, b2), (w3, b3) = layers
    dout = w3.shape[1]
    const = lambda s: pl.BlockSpec(s, lambda i: (0, 0))
    wcols = _CPW * K_NEIGH
    out = pl.pallas_call(
        _sa1_mlp_kernel,
        grid=(_NW,),
        in_specs=[
            pl.BlockSpec((6, wcols), lambda i: (0, i)),
            const((w1.shape[1], w1.shape[0])), const((w1.shape[1], 1)),
            const((w2.shape[1], w2.shape[0])), const((w2.shape[1], 1)),
            const((w3.shape[1], w3.shape[0])), const((w3.shape[1], 1)),
        ],
        out_specs=pl.BlockSpec((1, dout, _CPW), lambda i: (i, 0, 0)),
        out_shape=jax.ShapeDtypeStruct((_NW, dout, _CPW), F32),
    )(h_t, w1.T, b1.reshape(-1, 1), w2.T, b2.reshape(-1, 1),
      w3.T, b3.reshape(-1, 1))
    return out.transpose(0, 2, 1).reshape(_C1SC, dout)


def _fp_kernel(rpos_ref, skip_ref, ct_ref, featc_ref, w1_ref, b1_ref,
               w2_ref, b2_ref, out_ref):
    rb = rpos_ref[...]
    ct = ct_ref[...]
    n = ct.shape[1]
    rn = jnp.sum(rb * rb, axis=1, keepdims=True)
    cn = jnp.sum(ct * ct, axis=0, keepdims=True)
    dist = rn + cn - 2.0 * _mm(rb, ct)
    iota = jax.lax.broadcasted_iota(jnp.int32, (1, n), 1)
    wacc = jnp.zeros_like(dist)
    wsum = jnp.zeros_like(rn)
    for _ in range(3):
        m = jnp.min(dist, axis=1, keepdims=True)
        ohf = _argmin_oh(dist, iota)
        wi = 1.0 / (jnp.maximum(m, 0.0) + 1e-8)
        wacc = wacc + ohf * wi
        wsum = wsum + wi
        dist = dist + ohf * _MASK_BIG
    interp = _mm(wacc / wsum, featc_ref[...])
    h = jnp.concatenate([interp, skip_ref[...]], axis=1)
    h = jnp.maximum(_mm(h, w1_ref[...]) + b1_ref[...], 0.0)
    out_ref[...] = jnp.maximum(_mm(h, w2_ref[...]) + b2_ref[...], 0.0)


def _fp_call(rpos, skip, cand_t, featc, layers, blk):
    c = rpos.shape[0]
    (w1, b1), (w2, b2) = layers
    dout = w2.shape[1]
    const = lambda s: pl.BlockSpec(s, lambda i: (0, 0))
    return pl.pallas_call(
        _fp_kernel,
        grid=(c // blk,),
        in_specs=[
            pl.BlockSpec((blk, 3), lambda i: (i, 0)),
            pl.BlockSpec((blk, skip.shape[1]), lambda i: (i, 0)),
            const(cand_t.shape),
            const(featc.shape),
            const(w1.shape), const((1, b1.shape[0])),
            const(w2.shape), const((1, b2.shape[0])),
        ],
        out_specs=pl.BlockSpec((blk, dout), lambda i: (i, 0)),
        out_shape=jax.ShapeDtypeStruct((c, dout), F32),
    )(rpos, skip, cand_t, featc, w1, b1.reshape(1, -1), w2, b2.reshape(1, -1))


def _head_kernel(pos_ref, f0_ref, ws1, bs1, ws2, bs2, wz11, bz11, wz12, bz12,
                 wz21, bz21, wz22, bz22, ww1, bw1, ww2, bw2, zz_ref, ss_ref):
    pf = jnp.concatenate([pos_ref[...], f0_ref[...]], axis=1)

    def head(w1, b1, w2, b2):
        h = jnp.maximum(_mm(pf, w1[...]) + b1[...], 0.0)
        return _mm(h, w2[...]) + b2[...]

    s = jax.nn.sigmoid(head(ws1, bs1, ws2, bs2))
    z1 = head(wz11, bz11, wz12, bz12)
    z2 = head(wz21, bz21, wz22, bz22)
    w = head(ww1, bw1, ww2, bw2)
    zz_ref[...] = jnp.concatenate([z1, z2, s, w], axis=1)

    part = jnp.concatenate(
        [jnp.sum(z1 * z1, keepdims=True).reshape(1, 1),
         jnp.sum(z2 * z2, keepdims=True).reshape(1, 1)], axis=1)

    @pl.when(pl.program_id(0) == 0)
    def _():
        ss_ref[...] = jnp.zeros_like(ss_ref)

    ss_ref[...] += part


def _grasp_kernel(pos_ref, zz_ref, ss_ref, g_ref, sw_ref):
    contact = pos_ref[...]
    zz = zz_ref[...]
    z1 = zz[:, 0:3]
    z2 = zz[:, 3:6]
    s = zz[:, 6:7]
    w = zz[:, 7:8]

    base = z1 / jnp.sqrt(ss_ref[0, 0])
    inner = jnp.sum(base * z2, axis=1, keepdims=True)
    approach = (z2 - base * inner) / jnp.sqrt(ss_ref[0, 1])
    c0 = base / jnp.sqrt(jnp.sum(base * base, axis=1, keepdims=True))
    c2 = approach / jnp.sqrt(jnp.sum(approach * approach, axis=1,
                                     keepdims=True))
    y = jnp.concatenate([
        c2[:, 1:2] * c0[:, 2:3] - c2[:, 2:3] * c0[:, 1:2],
        c2[:, 2:3] * c0[:, 0:1] - c2[:, 0:1] * c0[:, 2:3],
        c2[:, 0:1] * c0[:, 1:2] - c2[:, 1:2] * c0[:, 0:1],
    ], axis=1)
    c1 = y / jnp.sqrt(jnp.sum(y * y, axis=1, keepdims=True))
    t = contact + (w * 0.5) * c0 - GRIPPER_DEPTH * c2

    nrows = contact.shape[0]
    cols = []
    for i in range(3):
        cols += [c0[:, i:i + 1], c1[:, i:i + 1], c2[:, i:i + 1], t[:, i:i + 1]]
    cols += [jnp.zeros((nrows, 3), F32), jnp.ones((nrows, 1), F32)]
    g_ref[...] = jnp.concatenate(cols, axis=1)
    sw_ref[...] = jnp.concatenate([s, w], axis=1)


def _head_call(pos, f0, params, blk=2000):
    flat = []
    for name in ('head_s', 'head_z1', 'head_z2', 'head_w'):
        (w1, b1), (w2, b2) = params[name]
        flat += [w1, b1.reshape(1, -1), w2, b2.reshape(1, -1)]
    n = pos.shape[0]
    const = lambda s: pl.BlockSpec(s, lambda i: (0, 0))
    wspecs = [const(a.shape) for a in flat]
    zz, ss = pl.pallas_call(
        _head_kernel,
        grid=(n // blk,),
        in_specs=[pl.BlockSpec((blk, 3), lambda i: (i, 0)),
                  pl.BlockSpec((blk, f0.shape[1]), lambda i: (i, 0))] + wspecs,
        out_specs=[pl.BlockSpec((blk, 8), lambda i: (i, 0)),
                   pl.BlockSpec((1, 2), lambda i: (0, 0))],
        out_shape=[jax.ShapeDtypeStruct((n, 8), F32),
                   jax.ShapeDtypeStruct((1, 2), F32)],
    )(pos, f0, *flat)
    return pl.pallas_call(
        _grasp_kernel,
        grid=(n // blk,),
        in_specs=[pl.BlockSpec((blk, 3), lambda i: (i, 0)),
                  pl.BlockSpec((blk, 8), lambda i: (i, 0)),
                  const((1, 2))],
        out_specs=[pl.BlockSpec((blk, 16), lambda i: (i, 0)),
                   pl.BlockSpec((blk, 2), lambda i: (i, 0))],
        out_shape=[jax.ShapeDtypeStruct((n, 16), F32),
                   jax.ShapeDtypeStruct((n, 2), F32)],
    )(pos, zz, ss)


def kernel(input_pcd, pos, batch, params):
    npad = NPAD - N_POINTS
    pos_pad = jnp.concatenate(
        [pos, jnp.full((npad, 3), 1e6, F32)], axis=0)
    feat_pad = jnp.concatenate(
        [input_pcd, jnp.zeros((npad, 3), F32)], axis=0)
    pos1 = pos[:C1 * 4:4]
    pos_t = pos_pad.T
    d2m = _d2_call(pos1, pos_t)
    h_t = _sc_sa1_knn(d2m, pos_pad, feat_pad)
    feat1_sc = _sa1_mlp_call(h_t, params['sa1'])
    table1 = jnp.concatenate([pos_pad, feat_pad], axis=1)
    feat1_tc = _sa_call(pos1[_C1SC:], pos_t, table1,
                        params['sa1'], blk=128)
    feat1 = jnp.concatenate([feat1_sc, feat1_tc], axis=0)

    pos1_t = pos1.T
    table2 = jnp.concatenate([pos1, feat1], axis=1)
    pos2 = pos1[:C2 * 4:4]
    feat2 = _sa_call(pos2, pos1_t, table2, params['sa2'], blk=128)

    f1 = _fp_call(pos1, feat1, pos2.T, feat2, params['fp1'], blk=256)
    f0 = _fp_call(pos, input_pcd, pos1_t, f1, params['fp0'], blk=400)

    g16, sw = _head_call(pos, f0, params)
    grasps = g16.reshape(N_POINTS, 4, 4)
    return grasps, sw[:, 0:1], sw[:, 1:2]


# speedup vs baseline: 1.6126x; 1.1167x over previous
import functools

import jax
import jax.numpy as jnp
from jax import lax
from jax.experimental import pallas as pl
from jax.experimental.pallas import tpu as pltpu
from jax.experimental.pallas import tpu_sc as plsc

F32 = jnp.float32
N_POINTS = 10000
NPAD = 10240
C1 = 2048
C2 = 512
K_NEIGH = 32
GRIPPER_DEPTH = 0.1034


def _mm(a, b):
    return jax.lax.dot_general(a, b, (((1,), (0,)), ((), ())),
                               preferred_element_type=F32)


_MASK_BIG = 1e30


def _argmin_oh(dist, iota):
    idx = jnp.argmin(dist, axis=1)
    return (iota == idx[:, None]).astype(F32)


def _sa_kernel(cpos_ref, pt_ref, table_ref, w1_ref, b1_ref, w2_ref, b2_ref,
               w3_ref, b3_ref, out_ref, hbuf_ref, *, k, feat_dim, blk):
    cb = cpos_ref[...]
    pt = pt_ref[...]
    table = table_ref[...]
    n = pt.shape[1]
    cn = jnp.sum(cb * cb, axis=1, keepdims=True)
    pn = jnp.sum(pt * pt, axis=0, keepdims=True)
    dist = cn + pn - 2.0 * _mm(cb, pt)
    iota = jax.lax.broadcasted_iota(jnp.int32, (1, n), 1)
    d = 3 + feat_dim
    cpad = jnp.concatenate([cb, jnp.zeros((blk, feat_dim), F32)], axis=1)

    def body(i, dist):
        ohf = _argmin_oh(dist, iota)
        g = _mm(ohf, table) - cpad
        hbuf_ref[pl.ds(i * blk, blk), :] = g
        return dist + ohf * _MASK_BIG

    jax.lax.fori_loop(0, k, body, dist)

    h = jnp.maximum(_mm(hbuf_ref[...], w1_ref[...]) + b1_ref[...], 0.0)
    h = jnp.maximum(_mm(h, w2_ref[...]) + b2_ref[...], 0.0)
    h = jnp.maximum(_mm(h, w3_ref[...]) + b3_ref[...], 0.0)
    out_ref[...] = jnp.max(h.reshape(k, blk, h.shape[1]), axis=0)


def _sa_call(cpos, cand_t, table, layers, blk):
    c = cpos.shape[0]
    feat_dim = table.shape[1] - 3
    (w1, b1), (w2, b2), (w3, b3) = layers
    dout = w3.shape[1]
    const = lambda s: pl.BlockSpec(s, lambda i: (0, 0))
    return pl.pallas_call(
        functools.partial(_sa_kernel, k=K_NEIGH, feat_dim=feat_dim, blk=blk),
        grid=(c // blk,),
        in_specs=[
            pl.BlockSpec((blk, 3), lambda i: (i, 0)),
            const(cand_t.shape),
            const(table.shape),
            const(w1.shape), const((1, b1.shape[0])),
            const(w2.shape), const((1, b2.shape[0])),
            const(w3.shape), const((1, b3.shape[0])),
        ],
        out_specs=pl.BlockSpec((blk, dout), lambda i: (i, 0)),
        out_shape=jax.ShapeDtypeStruct((c, dout), F32),
        scratch_shapes=[pltpu.VMEM((K_NEIGH * blk, feat_dim + 3), F32)],
    )(cpos, cand_t, table, w1, b1.reshape(1, -1), w2, b2.reshape(1, -1),
      w3, b3.reshape(1, -1))


_NC, _NS, _L = 2, 16, 16
_NW = _NC * _NS
_C1SC = 1280
_CPW = _C1SC // _NW
_NV = NPAD // _L
_SC_BIG = 1e30


def _sc_knn_kernel(d2h, xh, yh, zh, fxh, fyh, fzh, outh,
                   xv, yv, zv, fxv, fyv, fzv, dbuf, cv, civ, st, cmv,
                   selbuf):
    wid = lax.axis_index("s") * _NC + lax.axis_index("c")
    pltpu.sync_copy(xh, xv)
    pltpu.sync_copy(yh, yv)
    pltpu.sync_copy(zh, zv)
    pltpu.sync_copy(fxh, fxv)
    pltpu.sync_copy(fyh, fyv)
    pltpu.sync_copy(fzh, fzv)
    iota = lax.iota(jnp.int32, _L)
    big16 = jnp.full((_L,), _SC_BIG, F32)

    def center_body(i, c):
        cg = wid * _CPW + i
        p = cg * 4
        base = (p // _L) * _L
        lmf = jnp.where(iota == (p - base), 1.0, 0.0)
        cx = jnp.sum(xv[pl.ds(base, _L)] * lmf)
        cy = jnp.sum(yv[pl.ds(base, _L)] * lmf)
        cz = jnp.sum(zv[pl.ds(base, _L)] * lmf)

        pltpu.sync_copy(d2h.at[cg], dbuf)

        cmv[pl.ds(0, _L)] = big16
        cmv[pl.ds(_L, _L)] = big16

        def cm_body(j2, c2):
            ja = 2 * j2
            jb = ja + 1
            cmv[pl.ds(0, _L)] = jnp.minimum(cmv[pl.ds(0, _L)],
                                            dbuf[pl.ds(ja * _L, _L)])
            cmv[pl.ds(_L, _L)] = jnp.minimum(cmv[pl.ds(_L, _L)],
                                             dbuf[pl.ds(jb * _L, _L)])
            return c2

        lax.fori_loop(0, _NV // 2, cm_body, 0)
        thr = jnp.maximum(jnp.max(cmv[pl.ds(0, _L)]),
                          jnp.max(cmv[pl.ds(_L, _L)]))

        def comp_body(j, cnt):
            d2 = dbuf[pl.ds(j * _L, _L)]
            msk = d2 <= thr
            mi = jnp.where(msk, 1, 0)
            pre = plsc.cumsum(mi)
            offs = cnt + pre - mi
            plsc.store_scatter(cv, [offs], d2, mask=msk)
            plsc.store_scatter(civ, [offs], j * _L + iota, mask=msk)
            return cnt + jnp.sum(mi)

        cnt = lax.fori_loop(0, _NV, comp_body, 0)
        plsc.store_scatter(cv, [cnt + iota], big16)
        nvec = (cnt + _L - 1) // _L

        def ext_body(k, c3):
            def am_body(j, carry):
                bv, bp = carry
                v = cv[pl.ds(j * _L, _L)]
                m = jnp.min(v)
                upd = m < bv
                pos = j * _L + jnp.min(jnp.where(v == m, iota, _L))
                return (jnp.where(upd, m, bv), jnp.where(upd, pos, bp))

            _, bp = lax.fori_loop(0, nvec, am_body,
                                  (jnp.float32(_SC_BIG * 2), 0))
            bs = (bp // _L) * _L
            ln = bp - bs
            cv[pl.ds(bs, _L)] = jnp.where(iota == ln, _SC_BIG,
                                          cv[pl.ds(bs, _L)])
            gi = jnp.sum(civ[pl.ds(bs, _L)] * jnp.where(iota == ln, 1, 0))
            plsc.store_scatter(selbuf, [jnp.full((_L,), k, jnp.int32)],
                               jnp.full((_L,), gi, jnp.int32),
                               mask=iota == 0)
            return c3

        lax.fori_loop(0, K_NEIGH, ext_body, 0)

        wcols = _CPW * K_NEIGH
        for h in (0, 1):
            sel = selbuf[pl.ds(h * _L, _L)]
            cols = (h * _L + iota) * _CPW + i
            plsc.store_scatter(st, [cols],
                               plsc.load_gather(xv, [sel]) - cx)
            plsc.store_scatter(st, [1 * wcols + cols],
                               plsc.load_gather(yv, [sel]) - cy)
            plsc.store_scatter(st, [2 * wcols + cols],
                               plsc.load_gather(zv, [sel]) - cz)
            plsc.store_scatter(st, [3 * wcols + cols],
                               plsc.load_gather(fxv, [sel]))
            plsc.store_scatter(st, [4 * wcols + cols],
                               plsc.load_gather(fyv, [sel]))
            plsc.store_scatter(st, [5 * wcols + cols],
                               plsc.load_gather(fzv, [sel]))
        return c

    lax.fori_loop(0, _CPW, center_body, 0)
    for r in range(6):
        pltpu.sync_copy(
            st.at[pl.ds(r * _CPW * K_NEIGH, _CPW * K_NEIGH)],
            outh.at[pl.ds(r * _C1SC * K_NEIGH + wid * _CPW * K_NEIGH,
                          _CPW * K_NEIGH)])


def _d2_kernel(cpos_ref, pt_ref, out_ref):
    cb = cpos_ref[...]
    pt = pt_ref[...]
    cn = jnp.sum(cb * cb, axis=1, keepdims=True)
    pn = jnp.sum(pt * pt, axis=0, keepdims=True)
    out_ref[...] = cn + pn - 2.0 * _mm(cb, pt)


def _d2_call(cpos, pos_t):
    return pl.pallas_call(
        _d2_kernel,
        grid=(C1 // 128,),
        in_specs=[pl.BlockSpec((128, 3), lambda i: (i, 0)),
                  pl.BlockSpec(pos_t.shape, lambda i: (0, 0))],
        out_specs=pl.BlockSpec((128, NPAD), lambda i: (i, 0)),
        out_shape=jax.ShapeDtypeStruct((C1, NPAD), F32),
    )(cpos, pos_t)


def _sc_sa1_knn(d2m, pos_pad, feat_pad):
    mesh = plsc.VectorSubcoreMesh(core_axis_name="c", subcore_axis_name="s",
                                  num_cores=_NC, num_subcores=_NS)
    fn = pl.kernel(
        _sc_knn_kernel,
        out_type=jax.ShapeDtypeStruct((6 * _C1SC * K_NEIGH,), F32),
        mesh=mesh,
        compiler_params=pltpu.CompilerParams(needs_layout_passes=False),
        scratch_types=[pltpu.VMEM((NPAD,), F32)] * 6
        + [pltpu.VMEM((NPAD,), F32),
           pltpu.VMEM((NPAD + _L,), F32),
           pltpu.VMEM((NPAD + _L,), jnp.int32),
           pltpu.VMEM((6 * _CPW * K_NEIGH,), F32),
           pltpu.VMEM((2 * _L,), F32),
           pltpu.VMEM((K_NEIGH,), jnp.int32)],
    )
    out = fn(d2m, pos_pad[:, 0], pos_pad[:, 1], pos_pad[:, 2],
             feat_pad[:, 0], feat_pad[:, 1], feat_pad[:, 2])
    return out.reshape(6, _C1SC * K_NEIGH)


def _sa1_mlp_kernel(h_ref, w1_ref, b1_ref, w2_ref, b2_ref, w3_ref, b3_ref,
                    out_ref):
    a = jnp.maximum(_mm(w1_ref[...], h_ref[...]) + b1_ref[...], 0.0)
    a = jnp.maximum(_mm(w2_ref[...], a) + b2_ref[...], 0.0)
    a = jnp.maximum(_mm(w3_ref[...], a) + b3_ref[...], 0.0)
    m = a[:, 0:_CPW]
    for j in range(1, K_NEIGH):
        m = jnp.maximum(m, a[:, j * _CPW:(j + 1) * _CPW])
    out_ref[...] = m[None]


def _sa1_mlp_call(h_t, layers):
    (w1, b1), (w2, b2), (w3, b3) = layers
    dout = w3.shape[1]
    const = lambda s: pl.BlockSpec(s, lambda i: (0, 0))
    wcols = _CPW * K_NEIGH
    out = pl.pallas_call(
        _sa1_mlp_kernel,
        grid=(_NW,),
        in_specs=[
            pl.BlockSpec((6, wcols), lambda i: (0, i)),
            const((w1.shape[1], w1.shape[0])), const((w1.shape[1], 1)),
            const((w2.shape[1], w2.shape[0])), const((w2.shape[1], 1)),
            const((w3.shape[1], w3.shape[0])), const((w3.shape[1], 1)),
        ],
        out_specs=pl.BlockSpec((1, dout, _CPW), lambda i: (i, 0, 0)),
        out_shape=jax.ShapeDtypeStruct((_NW, dout, _CPW), F32),
    )(h_t, w1.T, b1.reshape(-1, 1), w2.T, b2.reshape(-1, 1),
      w3.T, b3.reshape(-1, 1))
    return out.transpose(0, 2, 1).reshape(_C1SC, dout)


def _fp_kernel(rpos_ref, skip_ref, ct_ref, featc_ref, w1_ref, b1_ref,
               w2_ref, b2_ref, out_ref):
    rb = rpos_ref[...]
    ct = ct_ref[...]
    n = ct.shape[1]
    rn = jnp.sum(rb * rb, axis=1, keepdims=True)
    cn = jnp.sum(ct * ct, axis=0, keepdims=True)
    dist = rn + cn - 2.0 * _mm(rb, ct)
    iota = jax.lax.broadcasted_iota(jnp.int32, (1, n), 1)
    wacc = jnp.zeros_like(dist)
    wsum = jnp.zeros_like(rn)
    for _ in range(3):
        m = jnp.min(dist, axis=1, keepdims=True)
        ohf = _argmin_oh(dist, iota)
        wi = 1.0 / (jnp.maximum(m, 0.0) + 1e-8)
        wacc = wacc + ohf * wi
        wsum = wsum + wi
        dist = dist + ohf * _MASK_BIG
    interp = _mm(wacc / wsum, featc_ref[...])
    h = jnp.concatenate([interp, skip_ref[...]], axis=1)
    h = jnp.maximum(_mm(h, w1_ref[...]) + b1_ref[...], 0.0)
    out_ref[...] = jnp.maximum(_mm(h, w2_ref[...]) + b2_ref[...], 0.0)


def _fp_call(rpos, skip, cand_t, featc, layers, blk):
    c = rpos.shape[0]
    (w1, b1), (w2, b2) = layers
    dout = w2.shape[1]
    const = lambda s: pl.BlockSpec(s, lambda i: (0, 0))
    return pl.pallas_call(
        _fp_kernel,
        grid=(c // blk,),
        in_specs=[
            pl.BlockSpec((blk, 3), lambda i: (i, 0)),
            pl.BlockSpec((blk, skip.shape[1]), lambda i: (i, 0)),
            const(cand_t.shape),
            const(featc.shape),
            const(w1.shape), const((1, b1.shape[0])),
            const(w2.shape), const((1, b2.shape[0])),
        ],
        out_specs=pl.BlockSpec((blk, dout), lambda i: (i, 0)),
        out_shape=jax.ShapeDtypeStruct((c, dout), F32),
    )(rpos, skip, cand_t, featc, w1, b1.reshape(1, -1), w2, b2.reshape(1, -1))


def _head_kernel(pos_ref, f0_ref, ws1, bs1, ws2, bs2, wz11, bz11, wz12, bz12,
                 wz21, bz21, wz22, bz22, ww1, bw1, ww2, bw2, zz_ref, ss_ref):
    pf = jnp.concatenate([pos_ref[...], f0_ref[...]], axis=1)

    def head(w1, b1, w2, b2):
        h = jnp.maximum(_mm(pf, w1[...]) + b1[...], 0.0)
        return _mm(h, w2[...]) + b2[...]

    s = jax.nn.sigmoid(head(ws1, bs1, ws2, bs2))
    z1 = head(wz11, bz11, wz12, bz12)
    z2 = head(wz21, bz21, wz22, bz22)
    w = head(ww1, bw1, ww2, bw2)
    zz_ref[...] = jnp.concatenate([z1, z2, s, w], axis=1)

    part = jnp.concatenate(
        [jnp.sum(z1 * z1, keepdims=True).reshape(1, 1),
         jnp.sum(z2 * z2, keepdims=True).reshape(1, 1)], axis=1)

    @pl.when(pl.program_id(0) == 0)
    def _():
        ss_ref[...] = jnp.zeros_like(ss_ref)

    ss_ref[...] += part


def _grasp_kernel(pos_ref, zz_ref, ss_ref, g_ref, sw_ref):
    contact = pos_ref[...]
    zz = zz_ref[...]
    z1 = zz[:, 0:3]
    z2 = zz[:, 3:6]
    s = zz[:, 6:7]
    w = zz[:, 7:8]

    base = z1 / jnp.sqrt(ss_ref[0, 0])
    inner = jnp.sum(base * z2, axis=1, keepdims=True)
    approach = (z2 - base * inner) / jnp.sqrt(ss_ref[0, 1])
    c0 = base / jnp.sqrt(jnp.sum(base * base, axis=1, keepdims=True))
    c2 = approach / jnp.sqrt(jnp.sum(approach * approach, axis=1,
                                     keepdims=True))
    y = jnp.concatenate([
        c2[:, 1:2] * c0[:, 2:3] - c2[:, 2:3] * c0[:, 1:2],
        c2[:, 2:3] * c0[:, 0:1] - c2[:, 0:1] * c0[:, 2:3],
        c2[:, 0:1] * c0[:, 1:2] - c2[:, 1:2] * c0[:, 0:1],
    ], axis=1)
    c1 = y / jnp.sqrt(jnp.sum(y * y, axis=1, keepdims=True))
    t = contact + (w * 0.5) * c0 - GRIPPER_DEPTH * c2

    nrows = contact.shape[0]
    cols = []
    for i in range(3):
        cols += [c0[:, i:i + 1], c1[:, i:i + 1], c2[:, i:i + 1], t[:, i:i + 1]]
    cols += [jnp.zeros((nrows, 3), F32), jnp.ones((nrows, 1), F32)]
    g_ref[...] = jnp.concatenate(cols, axis=1)
    sw_ref[...] = jnp.concatenate([s, w], axis=1)


def _head_call(pos, f0, params, blk=2000):
    flat = []
    for name in ('head_s', 'head_z1', 'head_z2', 'head_w'):
        (w1, b1), (w2, b2) = params[name]
        flat += [w1, b1.reshape(1, -1), w2, b2.reshape(1, -1)]
    n = pos.shape[0]
    const = lambda s: pl.BlockSpec(s, lambda i: (0, 0))
    wspecs = [const(a.shape) for a in flat]
    zz, ss = pl.pallas_call(
        _head_kernel,
        grid=(n // blk,),
        in_specs=[pl.BlockSpec((blk, 3), lambda i: (i, 0)),
                  pl.BlockSpec((blk, f0.shape[1]), lambda i: (i, 0))] + wspecs,
        out_specs=[pl.BlockSpec((blk, 8), lambda i: (i, 0)),
                   pl.BlockSpec((1, 2), lambda i: (0, 0))],
        out_shape=[jax.ShapeDtypeStruct((n, 8), F32),
                   jax.ShapeDtypeStruct((1, 2), F32)],
    )(pos, f0, *flat)
    return pl.pallas_call(
        _grasp_kernel,
        grid=(n // blk,),
        in_specs=[pl.BlockSpec((blk, 3), lambda i: (i, 0)),
                  pl.BlockSpec((blk, 8), lambda i: (i, 0)),
                  const((1, 2))],
        out_specs=[pl.BlockSpec((blk, 16), lambda i: (i, 0)),
                   pl.BlockSpec((blk, 2), lambda i: (i, 0))],
        out_shape=[jax.ShapeDtypeStruct((n, 16), F32),
                   jax.ShapeDtypeStruct((n, 2), F32)],
    )(pos, zz, ss)


def kernel(input_pcd, pos, batch, params):
    npad = NPAD - N_POINTS
    pos_pad = jnp.concatenate(
        [pos, jnp.full((npad, 3), 1e6, F32)], axis=0)
    feat_pad = jnp.concatenate(
        [input_pcd, jnp.zeros((npad, 3), F32)], axis=0)
    pos1 = pos[:C1 * 4:4]
    pos_t = pos_pad.T
    d2m = _d2_call(pos1, pos_t)
    h_t = _sc_sa1_knn(d2m, pos_pad, feat_pad)
    feat1_sc = _sa1_mlp_call(h_t, params['sa1'])
    table1 = jnp.concatenate([pos_pad, feat_pad], axis=1)
    feat1_tc = _sa_call(pos1[_C1SC:], pos_t, table1,
                        params['sa1'], blk=128)
    feat1 = jnp.concatenate([feat1_sc, feat1_tc], axis=0)

    pos1_t = pos1.T
    table2 = jnp.concatenate([pos1, feat1], axis=1)
    pos2 = pos1[:C2 * 4:4]
    feat2 = _sa_call(pos2, pos1_t, table2, params['sa2'], blk=128)

    f1 = _fp_call(pos1, feat1, pos2.T, feat2, params['fp1'], blk=256)
    f0 = _fp_call(pos, input_pcd, pos1_t, f1, params['fp0'], blk=400)

    g16, sw = _head_call(pos, f0, params)
    grasps = g16.reshape(N_POINTS, 4, 4)
    return grasps, sw[:, 0:1], sw[:, 1:2]
